# Initial kernel scaffold; baseline (speedup 1.0000x reference)
import jax
import jax.numpy as jnp
from jax.experimental import pallas as pl


def kernel(x, expert_ids, gate_weight, up_weight, down_weight):
    raise NotImplementedError("write your pallas kernel here")


# speedup vs baseline: 25.4142x; 25.4142x over previous
import jax
import jax.numpy as jnp
from jax.experimental import pallas as pl

NUM_EXPERTS = 16


def _moe_body(eids_ref, x_ref, gw_ref, uw_ref, dw_ref, out_ref):
    e = pl.program_id(0)
    x = x_ref[...]
    gw = gw_ref[0]
    uw = uw_ref[0]
    dw = dw_ref[0]
    gate = jax.lax.dot_general(x, gw, (((1,), (1,)), ((), ())),
                               preferred_element_type=jnp.float32)
    up = jax.lax.dot_general(x, uw, (((1,), (1,)), ((), ())),
                             preferred_element_type=jnp.float32)
    h = gate * jax.nn.sigmoid(gate) * up
    outp = jax.lax.dot_general(h, dw, (((1,), (1,)), ((), ())),
                               preferred_element_type=jnp.float32)
    mask = eids_ref[...] == e
    contrib = jnp.where(mask, outp, 0.0)

    @pl.when(e == 0)
    def _():
        out_ref[...] = contrib

    @pl.when(e > 0)
    def _():
        out_ref[...] += contrib


def kernel(x, expert_ids, gate_weight, up_weight, down_weight):
    n, d = x.shape
    num_e, hidden, _ = gate_weight.shape
    eids = expert_ids.reshape(n, 1)
    return pl.pallas_call(
        _moe_body,
        grid=(num_e,),
        in_specs=[
            pl.BlockSpec((n, 1), lambda e: (0, 0)),
            pl.BlockSpec((n, d), lambda e: (0, 0)),
            pl.BlockSpec((1, hidden, d), lambda e: (e, 0, 0)),
            pl.BlockSpec((1, hidden, d), lambda e: (e, 0, 0)),
            pl.BlockSpec((1, d, hidden), lambda e: (e, 0, 0)),
        ],
        out_specs=pl.BlockSpec((n, d), lambda e: (0, 0)),
        out_shape=jax.ShapeDtypeStruct((n, d), jnp.float32),
    )(eids, x, gate_weight, up_weight, down_weight)
